# baseline (speedup 1.0000x reference)
import functools

import numpy as np
import jax
import jax.numpy as jnp
from jax import lax
from jax.experimental import pallas as pl
from jax.experimental.pallas import tpu as pltpu
from jax.experimental.pallas import tpu_sc as plsc

_NC = 2
_NS = 16
_NW = _NC * _NS
_L = 16

_ROWPC = 14
_PIECES = ((0, 64), (64, 64), (128, 64), (192, 4))


def _bb_pos_table(qq, kk):
    strides = int(np.ceil(np.sqrt(float(kk) / float(qq))))
    qh = int(np.sqrt(float(qq)))
    kh = int(np.sqrt(float(kk)))
    x1, y1 = np.meshgrid(np.arange(qh), np.arange(qh))
    aa = np.stack([x1.reshape(-1), y1.reshape(-1)], axis=-1)
    x2, y2 = np.meshgrid(np.arange(kh), np.arange(kh))
    bbc = np.stack([x2.reshape(-1), y2.reshape(-1)], axis=-1)
    cc = np.abs(bbc[None, :, :] - aa[:, None, :] * strides)
    return (cc[:, :, 0] + cc[:, :, 1] * kh).astype(np.int32)


def kernel(inputs, bb):
    B, H, QQ, KK = inputs.shape
    assert (B, H, QQ, KK) == (128, 8, 196, 196)
    wph = _NW // H
    bpw = B // wph
    kpad = 256
    nck = KK // _L
    tl0 = KK - _L
    tzero = _L - (KK - nck * _L)
    tslot = (nck + 1) * _L

    idx_np = np.zeros((QQ, kpad), dtype=np.int32)
    idx_np[:, :KK] = _bb_pos_table(QQ, KK)
    idxp = jnp.asarray(idx_np.reshape(QQ // _ROWPC, _ROWPC, kpad))

    mesh = plsc.VectorSubcoreMesh(core_axis_name="c", subcore_axis_name="s")

    @functools.partial(
        pl.kernel,
        mesh=mesh,
        out_type=jax.ShapeDtypeStruct((B, H, QQ, KK), jnp.float32),
        scratch_types=[
            pltpu.VMEM((4, 64, KK), jnp.float32),
            pltpu.VMEM((QQ, tslot + _L), jnp.float32),
            pltpu.VMEM((_ROWPC, kpad), jnp.int32),
            pltpu.VMEM((bb.size,), jnp.float32),
            pltpu.SemaphoreType.DMA,
            pltpu.SemaphoreType.DMA,
            pltpu.SemaphoreType.DMA,
            pltpu.SemaphoreType.DMA,
            pltpu.SemaphoreType.DMA,
            pltpu.SemaphoreType.DMA,
            pltpu.SemaphoreType.DMA,
            pltpu.SemaphoreType.DMA,
        ],
        compiler_params=pltpu.CompilerParams(
            needs_layout_passes=False, use_tc_tiling_on_sc=True),
    )
    def sc_kernel(x_hbm, bbf_hbm, idx_hbm, o_hbm, io_v, pos_v, idx_v, bb_v,
                  is0, is1, is2, is3, os0, os1, os2, os3):
        insems = [is0, is1, is2, is3]
        outsems = [os0, os1, os2, os3]
        wid = lax.axis_index("s") * _NC + lax.axis_index("c")
        h = wid // wph
        r = wid % wph

        pltpu.sync_copy(bbf_hbm, bb_v)
        for pi in range(QQ // _ROWPC):
            pltpu.sync_copy(idx_hbm.at[pi], idx_v)

            def grow(rr, carry, pi=pi):
                for c in range(nck):
                    rows = idx_v[rr, pl.ds(c * _L, _L)]
                    vals = plsc.load_gather(bb_v, [rows * H + h])
                    pos_v[pi * _ROWPC + rr, pl.ds(c * _L, _L)] = vals
                rows_t = idx_v[rr, pl.ds(tl0, _L)]
                vals_t = plsc.load_gather(bb_v, [rows_t * H + h])
                lane = lax.iota(jnp.int32, _L)
                pos_v[pi * _ROWPC + rr, pl.ds(tslot, _L)] = jnp.where(
                    lane >= tzero, vals_t, 0.0)
                return carry

            lax.fori_loop(0, _ROWPC, grow, 0)

        def in_copy(b, p):
            off, nr = _PIECES[p]
            return pltpu.make_async_copy(
                x_hbm.at[b, h, pl.ds(off, nr), :],
                io_v.at[p, pl.ds(0, nr), :],
                insems[p])

        def out_copy(b, p):
            off, nr = _PIECES[p]
            return pltpu.make_async_copy(
                io_v.at[p, pl.ds(0, nr), :],
                o_hbm.at[b, h, pl.ds(off, nr), :],
                outsems[p])

        b0 = r * bpw
        for p in range(4):
            in_copy(b0, p).start()

        def plane(j, carry):
            b = b0 + j
            for p in range(4):
                if p == 0:
                    @pl.when(j > 0)
                    def _():
                        out_copy(b - 1, 3).wait()
                        in_copy(b, 3).start()
                in_copy(b, p).wait()

                off, nr = _PIECES[p]

                def add_row(rr, c2, p=p, off=off):
                    for c in range(nck):
                        plsc.addupdate(
                            io_v.at[p, rr, pl.ds(c * _L, _L)],
                            pos_v[off + rr, pl.ds(c * _L, _L)])
                    plsc.addupdate(
                        io_v.at[p, rr, pl.ds(tl0, _L)],
                        pos_v[off + rr, pl.ds(tslot, _L)])
                    return c2

                lax.fori_loop(0, nr, add_row, 0)
                out_copy(b, p).start()
                if p > 0:
                    out_copy(b, p - 1).wait()

                    @pl.when(j < bpw - 1)
                    def _():
                        in_copy(b + 1, p - 1).start()
            return carry

        lax.fori_loop(0, bpw, plane, 0)
        out_copy(b0 + bpw - 1, 3).wait()

    return sc_kernel(inputs, bb.reshape(-1), idxp)


# speedup vs baseline: 1.2447x; 1.2447x over previous
import functools

import numpy as np
import jax
import jax.numpy as jnp
from jax import lax
from jax.experimental import pallas as pl
from jax.experimental.pallas import tpu as pltpu
from jax.experimental.pallas import tpu_sc as plsc

_NC = 2
_NS = 16
_NW = _NC * _NS
_L = 16


def _bb_pos_table(qq, kk):
    strides = int(np.ceil(np.sqrt(float(kk) / float(qq))))
    qh = int(np.sqrt(float(qq)))
    kh = int(np.sqrt(float(kk)))
    x1, y1 = np.meshgrid(np.arange(qh), np.arange(qh))
    aa = np.stack([x1.reshape(-1), y1.reshape(-1)], axis=-1)
    x2, y2 = np.meshgrid(np.arange(kh), np.arange(kh))
    bbc = np.stack([x2.reshape(-1), y2.reshape(-1)], axis=-1)
    cc = np.abs(bbc[None, :, :] - aa[:, None, :] * strides)
    return (cc[:, :, 0] + cc[:, :, 1] * kh).astype(np.int32)


def _sc_gather(bb_flat, idx_pad, num_heads, n_pad):
    wph = _NW // num_heads
    cpw = n_pad // (wph * _L)
    span = cpw * _L

    mesh = plsc.VectorSubcoreMesh(core_axis_name="c", subcore_axis_name="s")

    @functools.partial(
        pl.kernel,
        mesh=mesh,
        out_type=jax.ShapeDtypeStruct((num_heads * n_pad,), jnp.float32),
        scratch_types=[
            pltpu.VMEM((span,), jnp.int32),
            pltpu.VMEM((span,), jnp.float32),
            pltpu.VMEM(bb_flat.shape, jnp.float32),
        ],
        compiler_params=pltpu.CompilerParams(needs_layout_passes=False),
    )
    def gather_kernel(bb_hbm, idx_hbm, out_hbm, idx_v, out_v, bb_v):
        wid = lax.axis_index("s") * _NC + lax.axis_index("c")
        h = wid // wph
        start = (wid % wph) * span
        pltpu.sync_copy(bb_hbm, bb_v)
        pltpu.sync_copy(idx_hbm.at[pl.ds(start, span)], idx_v)
        col = jnp.full((_L,), h, dtype=jnp.int32)

        def body(i, carry):
            off = pl.multiple_of(i * _L, _L)
            rows = idx_v[pl.ds(off, _L)] * num_heads + col
            out_v[pl.ds(off, _L)] = plsc.load_gather(bb_v, [rows])
            return carry

        lax.fori_loop(0, cpw, body, 0)
        out_off = pl.multiple_of(h * n_pad + start, 8)
        pltpu.sync_copy(out_v, out_hbm.at[pl.ds(out_off, span)])

    return gather_kernel(bb_flat, idx_pad)


def _add_body(x_ref, p_ref, o_ref):
    o_ref[...] = x_ref[...] + p_ref[...]


def kernel(inputs, bb):
    B, H, QQ, KK = inputs.shape
    n = QQ * KK

    wph = _NW // H
    cpw = -(-n // (wph * _L))
    n_pad = cpw * _L * wph
    idx_flat = np.zeros((n_pad,), dtype=np.int32)
    idx_flat[:n] = _bb_pos_table(QQ, KK).reshape(-1)

    pos_pad = _sc_gather(bb.reshape(-1), jnp.asarray(idx_flat), H, n_pad)
    pos = pos_pad.reshape(H, n_pad)[:, :n].reshape(H, QQ, KK)

    bblk = 4
    return pl.pallas_call(
        _add_body,
        grid=(B // bblk,),
        in_specs=[
            pl.BlockSpec((bblk, H, QQ, KK), lambda b: (b, 0, 0, 0)),
            pl.BlockSpec((H, QQ, KK), lambda b: (0, 0, 0)),
        ],
        out_specs=pl.BlockSpec((bblk, H, QQ, KK), lambda b: (b, 0, 0, 0)),
        out_shape=jax.ShapeDtypeStruct((B, H, QQ, KK), jnp.float32),
    )(inputs, pos)
